# baseline (speedup 1.0000x reference)
import functools

import jax
import jax.numpy as jnp
from jax.experimental import pallas as pl
from jax.experimental.pallas import tpu as pltpu

_BLK_T = 256
_BLK_N = 256


def _make_kernel(n_tri_steps, n_node_steps):
    last = n_tri_steps + n_node_steps - 1

    def scone_kernel(x_ref, b1_ref, b2_ref, w0_ref, w1_ref, w2_ref,
                     o_ref, ft_ref, *_unused):
        j = pl.program_id(0)

        @pl.when(j == 0)
        def _prep():
            xb = x_ref[...]
            ft_ref[...] = jax.lax.dot_general(
                w2_ref[...], xb, (((0,), (1,)), ((), ())),
                preferred_element_type=jnp.float32)
            o_ref[...] = jnp.dot(xb, w1_ref[...],
                                 preferred_element_type=jnp.float32)

        @pl.when(j < n_tri_steps)
        def _tri_step():
            b2b = b2_ref[...]
            tt = jnp.dot(ft_ref[...], b2b,
                         preferred_element_type=jnp.float32)
            o_ref[...] += jax.lax.dot_general(
                b2b, tt, (((1,), (1,)), ((), ())),
                preferred_element_type=jnp.float32)

        @pl.when(j >= n_tri_steps)
        def _node_step():
            b1b = b1_ref[...]
            u = jnp.dot(b1b, x_ref[...], preferred_element_type=jnp.float32)
            n = jnp.dot(u, w0_ref[...], preferred_element_type=jnp.float32)
            d0bt = jax.lax.dot_general(n, b1b, (((0,), (0,)), ((), ())),
                                       preferred_element_type=jnp.float32)

            @pl.when(j == n_tri_steps)
            def _init():
                ft_ref[...] = d0bt

            @pl.when(j > n_tri_steps)
            def _acc():
                ft_ref[...] += d0bt

        @pl.when(j == last)
        def _epilogue():
            o_ref[...] = jnp.maximum(
                o_ref[...] + jnp.swapaxes(ft_ref[...], 0, 1), 0.0)

    return scone_kernel


@functools.partial(jax.jit, static_argnames=("interpret",))
def kernel(x, B1, B2, W0, W1, W2, interpret=False):
    n_edges, in_f = x.shape
    n_nodes = B1.shape[0]
    n_tri = B2.shape[1]
    out_f = W0.shape[1]
    n_tri_steps = n_tri // _BLK_T
    n_node_steps = n_nodes // _BLK_N
    t_s = n_tri_steps

    out = pl.pallas_call(
        _make_kernel(n_tri_steps, n_node_steps),
        grid=(n_tri_steps + n_node_steps,),
        in_specs=[
            pl.BlockSpec((n_edges, in_f), lambda j: (0, 0)),
            pl.BlockSpec((_BLK_N, n_edges),
                         lambda j: (jnp.maximum(j - t_s, 0), 0)),
            pl.BlockSpec((n_edges, _BLK_T),
                         lambda j: (0, jnp.minimum(j, t_s - 1))),
            pl.BlockSpec((in_f, out_f), lambda j: (0, 0)),
            pl.BlockSpec((in_f, out_f), lambda j: (0, 0)),
            pl.BlockSpec((in_f, out_f), lambda j: (0, 0)),
        ],
        out_specs=pl.BlockSpec((n_edges, out_f), lambda j: (0, 0)),
        out_shape=jax.ShapeDtypeStruct((n_edges, out_f), jnp.float32),
        scratch_shapes=[pltpu.VMEM((in_f, n_edges), jnp.float32)],
        compiler_params=pltpu.CompilerParams(
            vmem_limit_bytes=64 * 1024 * 1024),
        interpret=interpret,
    )(x, B1, B2, W0, W1, W2)
    return out


# speedup vs baseline: 1.0833x; 1.0833x over previous
import functools

import jax
import jax.numpy as jnp
from jax.experimental import pallas as pl
from jax.experimental.pallas import tpu as pltpu

_BLK_T = 512
_BLK_N = 256


def _make_kernel(n_tri_steps, n_node_steps):
    last = n_tri_steps + n_node_steps - 1

    def scone_kernel(x_ref, b1_ref, b2_ref, w0_ref, w12_ref,
                     o_ref, ft_ref):
        j = pl.program_id(0)

        @pl.when(j == 0)
        def _prep():
            xb = x_ref[...]
            w12 = w12_ref[...]
            ft_ref[...] = jax.lax.dot_general(
                w12[:, 128:], xb, (((0,), (1,)), ((), ())),
                preferred_element_type=jnp.float32)
            o_ref[...] = jnp.dot(xb, w12[:, :128],
                                 preferred_element_type=jnp.float32)

        @pl.when(j < n_tri_steps)
        def _tri_step():
            b2b = b2_ref[...]
            tt = jnp.dot(ft_ref[...], b2b,
                         preferred_element_type=jnp.float32)
            for h in range(2):
                o_ref[h * 4096:(h + 1) * 4096, :] += jax.lax.dot_general(
                    b2_ref[h * 4096:(h + 1) * 4096, :], tt,
                    (((1,), (1,)), ((), ())),
                    preferred_element_type=jnp.float32)

        @pl.when(j >= n_tri_steps)
        def _node_step():
            b1b = b1_ref[...]
            u = jnp.dot(b1b, x_ref[...], preferred_element_type=jnp.float32)
            n = jnp.dot(u, w0_ref[...], preferred_element_type=jnp.float32)
            for h in range(2):
                hs = slice(h * 4096, (h + 1) * 4096)
                d0bt = jax.lax.dot_general(
                    n, b1_ref[:, hs], (((0,), (0,)), ((), ())),
                    preferred_element_type=jnp.float32)

                @pl.when(j == n_tri_steps)
                def _init(d0bt=d0bt, hs=hs):
                    ft_ref[:, hs] = d0bt

                @pl.when(j > n_tri_steps)
                def _acc(d0bt=d0bt, hs=hs):
                    ft_ref[:, hs] += d0bt

        @pl.when(j == last)
        def _epilogue():
            o_ref[...] = jnp.maximum(
                o_ref[...] + jnp.swapaxes(ft_ref[...], 0, 1), 0.0)

    return scone_kernel


@functools.partial(jax.jit, static_argnames=("interpret",))
def kernel(x, B1, B2, W0, W1, W2, interpret=False):
    n_edges, in_f = x.shape
    n_nodes = B1.shape[0]
    n_tri = B2.shape[1]
    out_f = W0.shape[1]
    n_tri_steps = n_tri // _BLK_T
    n_node_steps = n_nodes // _BLK_N
    t_s = n_tri_steps
    w12 = jnp.concatenate([W1, W2], axis=1)

    out = pl.pallas_call(
        _make_kernel(n_tri_steps, n_node_steps),
        grid=(n_tri_steps + n_node_steps,),
        in_specs=[
            pl.BlockSpec((n_edges, in_f), lambda j: (0, 0)),
            pl.BlockSpec((_BLK_N, n_edges),
                         lambda j: (jnp.maximum(j - t_s, 0), 0)),
            pl.BlockSpec((n_edges, _BLK_T),
                         lambda j: (0, jnp.minimum(j, t_s - 1))),
            pl.BlockSpec((in_f, out_f), lambda j: (0, 0)),
            pl.BlockSpec((in_f, 2 * out_f), lambda j: (0, 0)),
        ],
        out_specs=pl.BlockSpec((n_edges, out_f), lambda j: (0, 0)),
        out_shape=jax.ShapeDtypeStruct((n_edges, out_f), jnp.float32),
        scratch_shapes=[pltpu.VMEM((in_f, n_edges), jnp.float32)],
        compiler_params=pltpu.CompilerParams(
            vmem_limit_bytes=64 * 1024 * 1024),
        interpret=interpret,
    )(x, B1, B2, W0, w12)
    return out


# speedup vs baseline: 1.0843x; 1.0010x over previous
import functools

import jax
import jax.numpy as jnp
from jax.experimental import pallas as pl
from jax.experimental.pallas import tpu as pltpu

_BLK_T = 512
_BLK_N = 512


def _tri_kernel(x_ref, b2_ref, w1_ref, w2_ref, acc_ref, xw2t_ref):
    j = pl.program_id(0)

    @pl.when(j == 0)
    def _prep():
        xb = x_ref[...]
        xw2t_ref[...] = jax.lax.dot_general(
            w2_ref[...], xb, (((0,), (1,)), ((), ())),
            preferred_element_type=jnp.float32)
        acc_ref[...] = jnp.dot(xb, w1_ref[...],
                               preferred_element_type=jnp.float32)

    b2b = b2_ref[...]
    tt = jnp.dot(xw2t_ref[...], b2b, preferred_element_type=jnp.float32)
    acc_ref[...] += jax.lax.dot_general(
        b2b, tt, (((1,), (1,)), ((), ())),
        preferred_element_type=jnp.float32)


def _node_kernel(x_ref, b1_ref, w0_ref, acc_ref, o_ref, d0t_ref):
    i = pl.program_id(0)
    b1b = b1_ref[...]
    u = jnp.dot(b1b, x_ref[...], preferred_element_type=jnp.float32)
    n = jnp.dot(u, w0_ref[...], preferred_element_type=jnp.float32)
    d0bt = jax.lax.dot_general(n, b1b, (((0,), (0,)), ((), ())),
                               preferred_element_type=jnp.float32)

    @pl.when(i == 0)
    def _init():
        d0t_ref[...] = d0bt

    @pl.when(i > 0)
    def _acc():
        d0t_ref[...] += d0bt

    @pl.when(i == pl.num_programs(0) - 1)
    def _epilogue():
        o_ref[...] = jnp.maximum(
            acc_ref[...] + jnp.swapaxes(d0t_ref[...], 0, 1), 0.0)


@functools.partial(jax.jit, static_argnames=("interpret",))
def kernel(x, B1, B2, W0, W1, W2, interpret=False):
    n_edges, in_f = x.shape
    n_nodes = B1.shape[0]
    n_tri = B2.shape[1]
    out_f = W0.shape[1]

    acc = pl.pallas_call(
        _tri_kernel,
        grid=(n_tri // _BLK_T,),
        in_specs=[
            pl.BlockSpec((n_edges, in_f), lambda j: (0, 0)),
            pl.BlockSpec((n_edges, _BLK_T), lambda j: (0, j)),
            pl.BlockSpec((in_f, out_f), lambda j: (0, 0)),
            pl.BlockSpec((in_f, out_f), lambda j: (0, 0)),
        ],
        out_specs=pl.BlockSpec((n_edges, out_f), lambda j: (0, 0)),
        out_shape=jax.ShapeDtypeStruct((n_edges, out_f), jnp.float32),
        scratch_shapes=[pltpu.VMEM((in_f, n_edges), jnp.float32)],
        interpret=interpret,
    )(x, B2, W1, W2)

    out = pl.pallas_call(
        _node_kernel,
        grid=(n_nodes // _BLK_N,),
        in_specs=[
            pl.BlockSpec((n_edges, in_f), lambda i: (0, 0)),
            pl.BlockSpec((_BLK_N, n_edges), lambda i: (i, 0)),
            pl.BlockSpec((in_f, out_f), lambda i: (0, 0)),
            pl.BlockSpec((n_edges, out_f), lambda i: (0, 0)),
        ],
        out_specs=pl.BlockSpec((n_edges, out_f), lambda i: (0, 0)),
        out_shape=jax.ShapeDtypeStruct((n_edges, out_f), jnp.float32),
        scratch_shapes=[pltpu.VMEM((out_f, n_edges), jnp.float32)],
        interpret=interpret,
    )(x, B1, W0, acc)
    return out
